# baseline (speedup 1.0000x reference)
import functools

import jax
import jax.numpy as jnp
from jax import lax
from jax.experimental import pallas as pl
from jax.experimental.pallas import tpu as pltpu
from jax.experimental.pallas import tpu_sc as plsc

_NUM_WORKERS = 32
_CHUNK = 512
_NBUF = 2
_LOOKAHEAD = 1


def kernel(inputs, embeddings):
    batch, seq = inputs.shape
    vocab, width = embeddings.shape
    n = batch * seq
    n_per_w = n // _NUM_WORKERS
    n_chunks = n_per_w // _CHUNK
    n_outer = n_chunks // _NBUF

    idx3 = inputs.reshape(_NUM_WORKERS, n_chunks, _CHUNK)
    mesh = plsc.VectorSubcoreMesh(core_axis_name="c", subcore_axis_name="s")

    @functools.partial(
        pl.kernel,
        mesh=mesh,
        out_type=jax.ShapeDtypeStruct((n, width), jnp.float32),
        scratch_types=(
            [pltpu.VMEM((n_chunks, _CHUNK), jnp.int32),
             pltpu.VMEM((_NBUF, _CHUNK, width), jnp.float32)]
            + [pltpu.SemaphoreType.DMA] * (2 * _NBUF)
        ),
        compiler_params=pltpu.CompilerParams(use_tc_tiling_on_sc=False),
    )
    def run(idx_hbm, table_hbm, out_hbm, idx_v, rows_v, *sems):
        gsems = sems[:_NBUF]
        ssems = sems[_NBUF:]
        wid = lax.axis_index("s") * 2 + lax.axis_index("c")
        base = wid * n_per_w
        pltpu.sync_copy(idx_hbm.at[wid], idx_v)

        def start_gather(b, j):
            pltpu.async_copy(table_hbm.at[idx_v.at[j]], rows_v.at[b], gsems[b])

        def wait_gather(b):
            pltpu.make_async_copy(
                table_hbm.at[idx_v.at[0]], rows_v.at[b], gsems[b]).wait()

        def start_store(b, j):
            pltpu.async_copy(
                rows_v.at[b], out_hbm.at[pl.ds(base + j * _CHUNK, _CHUNK)],
                ssems[b])

        def wait_store(b):
            pltpu.make_async_copy(
                rows_v.at[b], out_hbm.at[pl.ds(base, _CHUNK)], ssems[b]).wait()

        for c in range(_LOOKAHEAD):
            start_gather(c, c)

        for b in range(_NBUF):
            wait_gather(b)
            start_store(b, b)
            bg = (b + _LOOKAHEAD) % _NBUF
            if b >= _LOOKAHEAD:
                wait_store(bg)
            start_gather(bg, b + _LOOKAHEAD)

        def body(t, carry):
            j0 = t * _NBUF
            for b in range(_NBUF):
                wait_gather(b)
                start_store(b, j0 + b)
                bg = (b + _LOOKAHEAD) % _NBUF
                wait_store(bg)
                start_gather(bg, j0 + b + _LOOKAHEAD)
            return carry

        lax.fori_loop(1, n_outer - 1, body, 0)

        j0 = (n_outer - 1) * _NBUF
        for b in range(_NBUF):
            wait_gather(b)
            start_store(b, j0 + b)
            if b < _LOOKAHEAD:
                bg = (b + _LOOKAHEAD) % _NBUF
                wait_store(bg)
                start_gather(bg, j0 + b + _LOOKAHEAD)

        for b in range(_NBUF):
            wait_store(b)

    out = run(idx3, embeddings)
    return out.reshape(batch, seq, width)


# speedup vs baseline: 1.2220x; 1.2220x over previous
import functools

import jax
import jax.numpy as jnp
from jax import lax
from jax.experimental import pallas as pl
from jax.experimental.pallas import tpu as pltpu
from jax.experimental.pallas import tpu_sc as plsc

_NUM_WORKERS = 32
_CHUNK = 256
_NBUF = 2
_LOOKAHEAD = 1


def kernel(inputs, embeddings):
    batch, seq = inputs.shape
    vocab, width = embeddings.shape
    n = batch * seq
    n_per_w = n // _NUM_WORKERS
    n_chunks = n_per_w // _CHUNK
    n_outer = n_chunks // _NBUF

    emb128 = jnp.pad(embeddings, ((0, 0), (0, width)))
    idx_flat = inputs.reshape(n)
    mesh = plsc.VectorSubcoreMesh(core_axis_name="c", subcore_axis_name="s")

    @functools.partial(
        pl.kernel,
        mesh=mesh,
        out_type=jax.ShapeDtypeStruct((n, 2 * width), jnp.float32),
        scratch_types=(
            [pltpu.VMEM((n_per_w,), jnp.int32),
             pltpu.VMEM((_NBUF, _CHUNK, 2 * width), jnp.float32)]
            + [pltpu.SemaphoreType.DMA] * (2 * _NBUF)
        ),
    )
    def run(idx_hbm, table_hbm, out_hbm, idx_v, rows_v, *sems):
        gsems = sems[:_NBUF]
        ssems = sems[_NBUF:]
        wid = lax.axis_index("s") * 2 + lax.axis_index("c")
        base = wid * n_per_w
        pltpu.sync_copy(idx_hbm.at[pl.ds(base, n_per_w)], idx_v)

        def start_gather(b, j):
            pltpu.async_copy(
                table_hbm.at[idx_v.at[pl.ds(j * _CHUNK, _CHUNK)]],
                rows_v.at[b], gsems[b])

        def wait_gather(b):
            pltpu.make_async_copy(
                table_hbm.at[idx_v.at[pl.ds(0, _CHUNK)]],
                rows_v.at[b], gsems[b]).wait()

        def start_store(b, j):
            pltpu.async_copy(
                rows_v.at[b],
                out_hbm.at[pl.ds(base + j * _CHUNK, _CHUNK)],
                ssems[b])

        def wait_store(b):
            pltpu.make_async_copy(
                rows_v.at[b],
                out_hbm.at[pl.ds(base, _CHUNK)], ssems[b]).wait()

        for c in range(_LOOKAHEAD):
            start_gather(c, c)

        for b in range(_NBUF):
            wait_gather(b)
            start_store(b, b)
            bg = (b + _LOOKAHEAD) % _NBUF
            if b >= _LOOKAHEAD:
                wait_store(bg)
            start_gather(bg, b + _LOOKAHEAD)

        def body(t, carry):
            j0 = t * _NBUF
            for b in range(_NBUF):
                wait_gather(b)
                start_store(b, j0 + b)
                bg = (b + _LOOKAHEAD) % _NBUF
                wait_store(bg)
                start_gather(bg, j0 + b + _LOOKAHEAD)
            return carry

        lax.fori_loop(1, n_outer - 1, body, 0)

        j0 = (n_outer - 1) * _NBUF
        for b in range(_NBUF):
            wait_gather(b)
            start_store(b, j0 + b)
            if b < _LOOKAHEAD:
                bg = (b + _LOOKAHEAD) % _NBUF
                wait_store(bg)
                start_gather(bg, j0 + b + _LOOKAHEAD)

        for b in range(_NBUF):
            wait_store(b)

    out = run(idx_flat, emb128)
    return out[:, :width].reshape(batch, seq, width)
